# baseline (speedup 1.0000x reference)
import functools

import jax
import jax.numpy as jnp
from jax import lax
from jax.experimental import pallas as pl
from jax.experimental.pallas import tpu as pltpu
from jax.experimental.pallas import tpu_sc as plsc

NUM_NODES = 100000
NUM_ELS = 10000
N_EDGES = 32
BATCH = 128

NC = 2
NS = 16
NW = NC * NS
L = 16
NVEC = BATCH // L

W_MAIN = NUM_ELS // NW
TAIL = NUM_ELS - W_MAIN * NW
C = 4
IDXC = C * N_EDGES
NCHUNK = W_MAIN // C
NBUF = 3


def _make_prod_kernel():
    mesh = plsc.VectorSubcoreMesh(core_axis_name="c", subcore_axis_name="s")

    @functools.partial(
        pl.kernel,
        out_type=jax.ShapeDtypeStruct((NUM_ELS, BATCH), jnp.float32),
        mesh=mesh,
        scratch_types=[
            pltpu.VMEM((W_MAIN * N_EDGES,), jnp.int32),
            pltpu.VMEM((N_EDGES,), jnp.int32),
            pltpu.VMEM((NBUF, IDXC, BATCH), jnp.float32),
            pltpu.VMEM((W_MAIN, BATCH), jnp.float32),
            [pltpu.SemaphoreType.DMA] * NBUF,
        ],
    )
    def prod_kernel(node_hbm, cids_hbm, out_hbm, idx_v, tidx_v, gat_v, out_v,
                    sems):
        w = lax.axis_index("s") * NC + lax.axis_index("c")
        base = w * W_MAIN

        pltpu.sync_copy(cids_hbm.at[pl.ds(base * N_EDGES, W_MAIN * N_EDGES)],
                        idx_v)

        for p in range(NBUF - 1):
            pltpu.async_copy(node_hbm.at[idx_v.at[pl.ds(p * IDXC, IDXC)]],
                             gat_v.at[p], sems[p])

        @pl.loop(0, NCHUNK, step=NBUF)
        def _chunk_group(cstep):
            for b in range(NBUF):
                c = cstep + b
                nxt = c + NBUF - 1
                nb = (b + NBUF - 1) % NBUF

                @pl.when(nxt < NCHUNK)
                def _start_next():
                    pltpu.async_copy(
                        node_hbm.at[idx_v.at[pl.ds(nxt * IDXC, IDXC)]],
                        gat_v.at[nb], sems[nb])

                pltpu.make_async_copy(
                    node_hbm.at[idx_v.at[pl.ds(0, IDXC)]],
                    gat_v.at[b], sems[b]).wait()

                for r in range(C):
                    acc = [gat_v[b, r * N_EDGES, pl.ds(d * L, L)]
                           for d in range(NVEC)]
                    for j in range(1, N_EDGES):
                        for d in range(NVEC):
                            acc[d] = acc[d] + gat_v[b, r * N_EDGES + j,
                                                    pl.ds(d * L, L)]
                    row = c * C + r
                    for d in range(NVEC):
                        out_v[row, pl.ds(d * L, L)] = acc[d]

        pltpu.sync_copy(out_v, out_hbm.at[pl.ds(base, W_MAIN)])

        @pl.when(w < TAIL)
        def _tail():
            e = NW * W_MAIN + w
            pltpu.sync_copy(cids_hbm.at[pl.ds(e * N_EDGES, N_EDGES)], tidx_v)
            pltpu.async_copy(node_hbm.at[tidx_v],
                             gat_v.at[0].at[pl.ds(0, N_EDGES)], sems[0]).wait()
            for d in range(NVEC):
                acc = gat_v[0, 0, pl.ds(d * L, L)]
                for j in range(1, N_EDGES):
                    acc = acc + gat_v[0, j, pl.ds(d * L, L)]
                out_v[0, pl.ds(d * L, L)] = acc
            pltpu.sync_copy(out_v.at[0], out_hbm.at[e])

    return prod_kernel


_PROD_KERNEL = _make_prod_kernel()


def kernel(node_mars, element_mars, nids, cids):
    del element_mars, nids
    cids_flat = cids.astype(jnp.int32).reshape(-1)
    return _PROD_KERNEL(node_mars, cids_flat)


# speedup vs baseline: 1.8529x; 1.8529x over previous
import functools

import jax
import jax.numpy as jnp
from jax import lax
from jax.experimental import pallas as pl
from jax.experimental.pallas import tpu as pltpu
from jax.experimental.pallas import tpu_sc as plsc

NUM_NODES = 100000
NUM_ELS = 10000
N_EDGES = 32
BATCH = 128

NC = 2
NS = 16
NW = NC * NS
L = 16
NVEC = BATCH // L

W_MAIN = NUM_ELS // NW
TAIL = NUM_ELS - W_MAIN * NW
C = 4
IDXC = C * N_EDGES
NCHUNK = W_MAIN // C
NBUF = 3


def _make_prod_kernel():
    mesh = plsc.VectorSubcoreMesh(core_axis_name="c", subcore_axis_name="s")

    @functools.partial(
        pl.kernel,
        out_type=jax.ShapeDtypeStruct((NUM_ELS, BATCH), jnp.float32),
        mesh=mesh,
        scratch_types=[
            pltpu.VMEM((W_MAIN * N_EDGES,), jnp.int32),
            pltpu.VMEM((N_EDGES,), jnp.int32),
            pltpu.VMEM((NBUF, IDXC, BATCH), jnp.float32),
            pltpu.VMEM((W_MAIN, BATCH), jnp.float32),
            [pltpu.SemaphoreType.DMA] * NBUF,
        ],
    )
    def prod_kernel(node_hbm, cids_hbm, out_hbm, idx_v, tidx_v, gat_v, out_v,
                    sems):
        w = lax.axis_index("s") * NC + lax.axis_index("c")
        base = w * W_MAIN

        pltpu.sync_copy(cids_hbm.at[pl.ds(base * N_EDGES, W_MAIN * N_EDGES)],
                        idx_v)

        for p in range(NBUF - 1):
            pltpu.async_copy(node_hbm.at[idx_v.at[pl.ds(p * IDXC, IDXC)]],
                             gat_v.at[p], sems[p])

        @pl.loop(0, NCHUNK, step=NBUF)
        def _chunk_group(cstep):
            for b in range(NBUF):
                c = cstep + b
                nxt = c + NBUF - 1
                nb = (b + NBUF - 1) % NBUF

                @pl.when(nxt < NCHUNK)
                def _start_next():
                    pltpu.async_copy(
                        node_hbm.at[idx_v.at[pl.ds(nxt * IDXC, IDXC)]],
                        gat_v.at[nb], sems[nb])

                pltpu.make_async_copy(
                    node_hbm.at[idx_v.at[pl.ds(0, IDXC)]],
                    gat_v.at[b], sems[b]).wait()

                @pl.loop(0, C)
                def _row(r):
                    rowbase = r * N_EDGES
                    acc = [gat_v[b, rowbase, pl.ds(d * L, L)]
                           for d in range(NVEC)]
                    for j in range(1, N_EDGES):
                        for d in range(NVEC):
                            acc[d] = acc[d] + gat_v[b, rowbase + j,
                                                    pl.ds(d * L, L)]
                    row = c * C + r
                    for d in range(NVEC):
                        out_v[row, pl.ds(d * L, L)] = acc[d]

        pltpu.sync_copy(out_v, out_hbm.at[pl.ds(base, W_MAIN)])

        @pl.when(w < TAIL)
        def _tail():
            e = NW * W_MAIN + w
            pltpu.sync_copy(cids_hbm.at[pl.ds(e * N_EDGES, N_EDGES)], tidx_v)
            pltpu.async_copy(node_hbm.at[tidx_v],
                             gat_v.at[0].at[pl.ds(0, N_EDGES)], sems[0]).wait()
            for d in range(NVEC):
                acc = gat_v[0, 0, pl.ds(d * L, L)]
                for j in range(1, N_EDGES):
                    acc = acc + gat_v[0, j, pl.ds(d * L, L)]
                out_v[0, pl.ds(d * L, L)] = acc
            pltpu.sync_copy(out_v.at[0], out_hbm.at[e])

    return prod_kernel


_PROD_KERNEL = _make_prod_kernel()


def kernel(node_mars, element_mars, nids, cids):
    del element_mars, nids
    cids_flat = cids.astype(jnp.int32).reshape(-1)
    return _PROD_KERNEL(node_mars, cids_flat)


# speedup vs baseline: 2.0876x; 1.1267x over previous
import functools

import jax
import jax.numpy as jnp
from jax import lax
from jax.experimental import pallas as pl
from jax.experimental.pallas import tpu as pltpu
from jax.experimental.pallas import tpu_sc as plsc

NUM_NODES = 100000
NUM_ELS = 10000
N_EDGES = 32
BATCH = 128

NC = 2
NS = 16
NW = NC * NS
L = 16
NVEC = BATCH // L

W_MAIN = NUM_ELS // NW
TAIL = NUM_ELS - W_MAIN * NW
C = 4
IDXC = C * N_EDGES
NCHUNK = W_MAIN // C
NBUF = 3


def _make_prod_kernel():
    mesh = plsc.VectorSubcoreMesh(core_axis_name="c", subcore_axis_name="s")

    @functools.partial(
        pl.kernel,
        out_type=jax.ShapeDtypeStruct((NUM_ELS, BATCH), jnp.float32),
        mesh=mesh,
        scratch_types=[
            pltpu.VMEM((W_MAIN * N_EDGES,), jnp.int32),
            pltpu.VMEM((N_EDGES,), jnp.int32),
            pltpu.VMEM((NBUF, IDXC, BATCH), jnp.float32),
            pltpu.VMEM((W_MAIN, BATCH), jnp.float32),
            [pltpu.SemaphoreType.DMA] * NBUF,
        ],
    )
    def prod_kernel(node_hbm, cids_hbm, out_hbm, idx_v, tidx_v, gat_v, out_v,
                    sems):
        w = lax.axis_index("s") * NC + lax.axis_index("c")
        base = w * W_MAIN

        pltpu.sync_copy(cids_hbm.at[pl.ds(base * N_EDGES, W_MAIN * N_EDGES)],
                        idx_v)

        for p in range(NBUF - 1):
            pltpu.async_copy(node_hbm.at[idx_v.at[pl.ds(p * IDXC, IDXC)]],
                             gat_v.at[p], sems[p])

        @pl.loop(0, NCHUNK, step=NBUF)
        def _chunk_group(cstep):
            for b in range(NBUF):
                c = cstep + b
                nxt = c + NBUF - 1
                nb = (b + NBUF - 1) % NBUF

                @pl.when(nxt < NCHUNK)
                def _start_next():
                    pltpu.async_copy(
                        node_hbm.at[idx_v.at[pl.ds(nxt * IDXC, IDXC)]],
                        gat_v.at[nb], sems[nb])

                pltpu.make_async_copy(
                    node_hbm.at[idx_v.at[pl.ds(0, IDXC)]],
                    gat_v.at[b], sems[b]).wait()

                @pl.loop(0, C)
                def _row(r):
                    rowbase = r * N_EDGES
                    row = c * C + r
                    for d in range(NVEC):
                        sl = pl.ds(d * L, L)
                        vals = []
                        for j in range(0, N_EDGES, 2):
                            vals.append(gat_v[b, rowbase + j, sl]
                                        + gat_v[b, rowbase + j + 1, sl])
                        while len(vals) > 1:
                            vals = [vals[i] + vals[i + 1]
                                    for i in range(0, len(vals), 2)]
                        out_v[row, sl] = vals[0]

        pltpu.sync_copy(out_v, out_hbm.at[pl.ds(base, W_MAIN)])

        @pl.when(w < TAIL)
        def _tail():
            e = NW * W_MAIN + w
            pltpu.sync_copy(cids_hbm.at[pl.ds(e * N_EDGES, N_EDGES)], tidx_v)
            pltpu.async_copy(node_hbm.at[tidx_v],
                             gat_v.at[0].at[pl.ds(0, N_EDGES)], sems[0]).wait()
            for d in range(NVEC):
                acc = gat_v[0, 0, pl.ds(d * L, L)]
                for j in range(1, N_EDGES):
                    acc = acc + gat_v[0, j, pl.ds(d * L, L)]
                out_v[0, pl.ds(d * L, L)] = acc
            pltpu.sync_copy(out_v.at[0], out_hbm.at[e])

    return prod_kernel


_PROD_KERNEL = _make_prod_kernel()


def kernel(node_mars, element_mars, nids, cids):
    del element_mars, nids
    cids_flat = cids.astype(jnp.int32).reshape(-1)
    return _PROD_KERNEL(node_mars, cids_flat)
